# baseline (device time: 89728 ns/iter reference)
import jax
import jax.numpy as jnp
from jax import lax
from jax.experimental import pallas as pl
from jax.experimental.pallas import tpu as pltpu

N_DEV = 4
F8 = jnp.float8_e4m3fn


def kernel(x, w_mat, scale_x, scale_w):
    m, k_shard = x.shape
    k, n = w_mat.shape
    m_per = m // N_DEV
    n_blk = 2048
    n_steps = n // n_blk
    assert k == N_DEV * k_shard

    me_arr = jnp.full((1,), lax.axis_index("i"), jnp.int32)

    _CHUNK_OFFS = (3, 2, 1, 0)
    _SEND_IDX = {3: 0, 2: 1, 1: 2}

    def body(me_ref, x_ref, w_ref, sx_ref, sw_ref, out_ref,
             acc_ref, xg_ref, obuf_ref, xs_ref, stage_ref, send_sems,
             recv_sems, stage_sems, out_sems):
        qi = pl.program_id(0)
        ji = pl.program_id(1)
        me = me_ref[0]
        s = lax.rem(me + qi, N_DEV)

        def _chunk_dma(idx, off):
            d = lax.rem(me + off, N_DEV)
            return pltpu.make_async_copy(
                x_ref.at[pl.ds(d * m_per, m_per), :],
                stage_ref.at[idx % 2],
                stage_sems.at[idx % 2],
            )

        def _chunk_proc(idx, off):
            _chunk_dma(idx, off).wait()
            f8 = stage_ref[idx % 2].astype(F8)
            if off == 0:
                xg_ref[me] = f8
            else:
                cnt = _SEND_IDX[off]
                xs_ref[cnt] = f8
                pltpu.make_async_remote_copy(
                    src_ref=xs_ref.at[cnt],
                    dst_ref=xg_ref.at[me],
                    send_sem=send_sems.at[cnt],
                    recv_sem=recv_sems.at[me],
                    device_id=(lax.rem(me + off, N_DEV),),
                    device_id_type=pl.DeviceIdType.MESH,
                ).start()

        @pl.when((qi == 0) & (ji == 0))
        def _start_comm():
            _chunk_dma(0, _CHUNK_OFFS[0]).start()
            _chunk_dma(1, _CHUNK_OFFS[1]).start()

            barrier_sem = pltpu.get_barrier_semaphore()
            for h in range(1, N_DEV):
                pl.semaphore_signal(
                    barrier_sem, inc=1,
                    device_id=(lax.rem(me + h, N_DEV),),
                    device_id_type=pl.DeviceIdType.MESH,
                )
            pl.semaphore_wait(barrier_sem, N_DEV - 1)

            _chunk_proc(0, _CHUNK_OFFS[0])
            _chunk_dma(2, _CHUNK_OFFS[2]).start()
            _chunk_proc(1, _CHUNK_OFFS[1])
            _chunk_dma(3, _CHUNK_OFFS[3]).start()
            _chunk_proc(2, _CHUNK_OFFS[2])
            _chunk_proc(3, _CHUNK_OFFS[3])

        @pl.when((qi > 0) & (ji == 0))
        def _wait_block():
            pltpu.make_async_remote_copy(
                src_ref=xs_ref.at[0],
                dst_ref=xg_ref.at[s],
                send_sem=send_sems.at[0],
                recv_sem=recv_sems.at[s],
                device_id=(s,),
                device_id_type=pl.DeviceIdType.MESH,
            ).wait_recv()

        js = pl.ds(ji * n_blk, n_blk)
        scale = sx_ref[0] * sw_ref[0]
        p = jnp.dot(xg_ref[s], w_ref[...].astype(F8),
                    preferred_element_type=jnp.float32)

        @pl.when(qi == 0)
        def _():
            acc_ref[:, js] = p.astype(jnp.bfloat16)

        @pl.when((qi > 0) & (qi < N_DEV - 1))
        def _():
            acc_ref[:, js] = (acc_ref[:, js] + p).astype(jnp.bfloat16)

        @pl.when(qi == N_DEV - 1)
        def _():
            slot = lax.rem(ji, 2)

            @pl.when(ji >= 2)
            def _():
                pltpu.make_async_copy(
                    obuf_ref.at[slot],
                    out_ref.at[:, pl.ds((ji - 2) * n_blk, n_blk)],
                    out_sems.at[slot],
                ).wait()

            obuf_ref[slot] = jnp.maximum(
                (acc_ref[:, js] + p) * scale, 0.0
            ).astype(jnp.bfloat16)
            pltpu.make_async_copy(
                obuf_ref.at[slot], out_ref.at[:, js], out_sems.at[slot],
            ).start()

        @pl.when((qi == N_DEV - 1) & (ji == n_steps - 1))
        def _drain():
            for jj in (n_steps - 2, n_steps - 1):
                pltpu.make_async_copy(
                    obuf_ref.at[jj % 2],
                    out_ref.at[:, pl.ds(jj * n_blk, n_blk)],
                    out_sems.at[jj % 2],
                ).wait()
            for off, cnt in _SEND_IDX.items():
                pltpu.make_async_remote_copy(
                    src_ref=xs_ref.at[cnt],
                    dst_ref=xg_ref.at[me],
                    send_sem=send_sems.at[cnt],
                    recv_sem=recv_sems.at[me],
                    device_id=(lax.rem(me + off, N_DEV),),
                    device_id_type=pl.DeviceIdType.MESH,
                ).wait_send()

    grid_spec = pltpu.PrefetchScalarGridSpec(
        num_scalar_prefetch=1,
        grid=(N_DEV, n_steps),
        in_specs=[
            pl.BlockSpec(memory_space=pl.ANY),
            pl.BlockSpec(
                (k_shard, n_blk),
                lambda q, j, m: (lax.rem(m[0] + q, N_DEV), j),
            ),
            pl.BlockSpec(memory_space=pltpu.SMEM),
            pl.BlockSpec(memory_space=pltpu.SMEM),
        ],
        out_specs=pl.BlockSpec(memory_space=pl.ANY),
        scratch_shapes=[
            pltpu.VMEM((m_per, n), jnp.bfloat16),
            pltpu.VMEM((N_DEV, m_per, k_shard), F8),
            pltpu.VMEM((2, m_per, n_blk), jnp.bfloat16),
            pltpu.VMEM((N_DEV - 1, m_per, k_shard), F8),
            pltpu.VMEM((2, m_per, k_shard), jnp.float32),
            pltpu.SemaphoreType.DMA((N_DEV - 1,)),
            pltpu.SemaphoreType.DMA((N_DEV,)),
            pltpu.SemaphoreType.DMA((2,)),
            pltpu.SemaphoreType.DMA((2,)),
        ],
    )

    return pl.pallas_call(
        body,
        grid_spec=grid_spec,
        out_shape=jax.ShapeDtypeStruct((m_per, n), jnp.bfloat16),
        compiler_params=pltpu.CompilerParams(
            collective_id=0,
            dimension_semantics=("arbitrary", "arbitrary"),
            vmem_limit_bytes=100 * 1024 * 1024,
        ),
    )(me_arr, x, w_mat, scale_x, scale_w)


# device time: 89303 ns/iter; 1.0048x vs baseline; 1.0048x over previous
import jax
import jax.numpy as jnp
from jax import lax
from jax.experimental import pallas as pl
from jax.experimental.pallas import tpu as pltpu

N_DEV = 4
F8 = jnp.float8_e4m3fn


def kernel(x, w_mat, scale_x, scale_w):
    m, k_shard = x.shape
    k, n = w_mat.shape
    m_per = m // N_DEV
    n_blk = 2048
    n_steps = n // n_blk
    assert k == N_DEV * k_shard

    me_arr = jnp.full((1,), lax.axis_index("i"), jnp.int32)

    _CHUNK_OFFS = (3, 2, 1, 0)
    _SEND_IDX = {3: 0, 2: 1, 1: 2}

    def body(me_ref, x_ref, w_ref, sx_ref, sw_ref, out_ref,
             acc_ref, xg_ref, obuf_ref, xs_ref, stage_ref, send_sems,
             recv_sems, stage_sems, out_sems):
        qi = pl.program_id(0)
        ji = pl.program_id(1)
        me = me_ref[0]
        s = lax.rem(me + (qi ^ (qi >> 1)), N_DEV)

        def _chunk_dma(idx, off):
            d = lax.rem(me + off, N_DEV)
            return pltpu.make_async_copy(
                x_ref.at[pl.ds(d * m_per, m_per), :],
                stage_ref.at[idx % 2],
                stage_sems.at[idx % 2],
            )

        def _chunk_proc(idx, off):
            _chunk_dma(idx, off).wait()
            f8 = stage_ref[idx % 2].astype(F8)
            if off == 0:
                xg_ref[me] = f8
            else:
                cnt = _SEND_IDX[off]
                xs_ref[cnt] = f8
                pltpu.make_async_remote_copy(
                    src_ref=xs_ref.at[cnt],
                    dst_ref=xg_ref.at[me],
                    send_sem=send_sems.at[cnt],
                    recv_sem=recv_sems.at[me],
                    device_id=(lax.rem(me + off, N_DEV),),
                    device_id_type=pl.DeviceIdType.MESH,
                ).start()

        @pl.when((qi == 0) & (ji == 0))
        def _start_comm():
            _chunk_dma(0, _CHUNK_OFFS[0]).start()
            _chunk_dma(1, _CHUNK_OFFS[1]).start()

            barrier_sem = pltpu.get_barrier_semaphore()
            for h in range(1, N_DEV):
                pl.semaphore_signal(
                    barrier_sem, inc=1,
                    device_id=(lax.rem(me + h, N_DEV),),
                    device_id_type=pl.DeviceIdType.MESH,
                )
            pl.semaphore_wait(barrier_sem, N_DEV - 1)

            _chunk_proc(0, _CHUNK_OFFS[0])
            _chunk_dma(2, _CHUNK_OFFS[2]).start()
            _chunk_proc(1, _CHUNK_OFFS[1])
            _chunk_dma(3, _CHUNK_OFFS[3]).start()
            _chunk_proc(2, _CHUNK_OFFS[2])
            _chunk_proc(3, _CHUNK_OFFS[3])

        @pl.when((qi > 0) & (ji == 0))
        def _wait_block():
            pltpu.make_async_remote_copy(
                src_ref=xs_ref.at[0],
                dst_ref=xg_ref.at[s],
                send_sem=send_sems.at[0],
                recv_sem=recv_sems.at[s],
                device_id=(s,),
                device_id_type=pl.DeviceIdType.MESH,
            ).wait_recv()

        js = pl.ds(ji * n_blk, n_blk)
        scale = sx_ref[0] * sw_ref[0]
        p = jnp.dot(xg_ref[s], w_ref[...].astype(F8),
                    preferred_element_type=jnp.float32)

        @pl.when(qi == 0)
        def _():
            acc_ref[:, js] = p.astype(jnp.bfloat16)

        @pl.when((qi > 0) & (qi < N_DEV - 1))
        def _():
            acc_ref[:, js] = (acc_ref[:, js] + p).astype(jnp.bfloat16)

        @pl.when(qi == N_DEV - 1)
        def _():
            slot = lax.rem(ji, 2)

            @pl.when(ji >= 2)
            def _():
                pltpu.make_async_copy(
                    obuf_ref.at[slot],
                    out_ref.at[:, pl.ds((ji - 2) * n_blk, n_blk)],
                    out_sems.at[slot],
                ).wait()

            obuf_ref[slot] = jnp.maximum(
                (acc_ref[:, js] + p) * scale, 0.0
            ).astype(jnp.bfloat16)
            pltpu.make_async_copy(
                obuf_ref.at[slot], out_ref.at[:, js], out_sems.at[slot],
            ).start()

        @pl.when((qi == N_DEV - 1) & (ji == n_steps - 1))
        def _drain():
            for jj in (n_steps - 2, n_steps - 1):
                pltpu.make_async_copy(
                    obuf_ref.at[jj % 2],
                    out_ref.at[:, pl.ds(jj * n_blk, n_blk)],
                    out_sems.at[jj % 2],
                ).wait()
            for off, cnt in _SEND_IDX.items():
                pltpu.make_async_remote_copy(
                    src_ref=xs_ref.at[cnt],
                    dst_ref=xg_ref.at[me],
                    send_sem=send_sems.at[cnt],
                    recv_sem=recv_sems.at[me],
                    device_id=(lax.rem(me + off, N_DEV),),
                    device_id_type=pl.DeviceIdType.MESH,
                ).wait_send()

    grid_spec = pltpu.PrefetchScalarGridSpec(
        num_scalar_prefetch=1,
        grid=(N_DEV, n_steps),
        in_specs=[
            pl.BlockSpec(memory_space=pl.ANY),
            pl.BlockSpec(
                (k_shard, n_blk),
                lambda q, j, m: (lax.rem(m[0] + (q ^ (q >> 1)), N_DEV), j),
            ),
            pl.BlockSpec(memory_space=pltpu.SMEM),
            pl.BlockSpec(memory_space=pltpu.SMEM),
        ],
        out_specs=pl.BlockSpec(memory_space=pl.ANY),
        scratch_shapes=[
            pltpu.VMEM((m_per, n), jnp.bfloat16),
            pltpu.VMEM((N_DEV, m_per, k_shard), F8),
            pltpu.VMEM((2, m_per, n_blk), jnp.bfloat16),
            pltpu.VMEM((N_DEV - 1, m_per, k_shard), F8),
            pltpu.VMEM((2, m_per, k_shard), jnp.float32),
            pltpu.SemaphoreType.DMA((N_DEV - 1,)),
            pltpu.SemaphoreType.DMA((N_DEV,)),
            pltpu.SemaphoreType.DMA((2,)),
            pltpu.SemaphoreType.DMA((2,)),
        ],
    )

    return pl.pallas_call(
        body,
        grid_spec=grid_spec,
        out_shape=jax.ShapeDtypeStruct((m_per, n), jnp.bfloat16),
        compiler_params=pltpu.CompilerParams(
            collective_id=0,
            dimension_semantics=("arbitrary", "arbitrary"),
            vmem_limit_bytes=100 * 1024 * 1024,
        ),
    )(me_arr, x, w_mat, scale_x, scale_w)
